# baseline (speedup 1.0000x reference)
import functools

import jax
import jax.numpy as jnp
from jax import lax
from jax.experimental import pallas as pl
from jax.experimental.pallas import tpu as pltpu
from jax.experimental.pallas import tpu_sc as plsc

B, N, D, K = 4, 8192, 2048, 256
BN = 256
NSTEPS = N // BN
CAND = 512
PADI = 0x3FFFFFFF
REPAIR_PASSES = 16


def _first(va, ia, vb, ib):
    return (va > vb) | ((va == vb) & (ia < ib))


def _rolls(x, d):
    L = x.shape[1]
    return pltpu.roll(x, L - d, 1), pltpu.roll(x, d, 1)


def _stage(v, i, d, keep_high, lanes):
    bit = (lanes & d) != 0
    vl, vr = _rolls(v, d)
    pv = jnp.where(bit, vr, vl)
    il, ir = _rolls(i, d)
    pi = jnp.where(bit, ir, il)
    g = _first(v, i, pv, pi)
    keep = g == keep_high
    return jnp.where(keep, v, pv), jnp.where(keep, i, pi)


def _sort_rev(v, i):
    L = v.shape[1]
    lanes = lax.broadcasted_iota(jnp.int32, v.shape, 1)
    k = 2
    while k <= L:
        d = k // 2
        while d >= 1:
            keep_high = ((lanes & d) == 0) != ((lanes & k) == 0)
            v, i = _stage(v, i, d, keep_high, lanes)
            d //= 2
        k *= 2
    return v, i


def _halfclean_desc(v, i):
    L = v.shape[1]
    lanes = lax.broadcasted_iota(jnp.int32, v.shape, 1)
    d = L // 2
    while d >= 1:
        keep_high = (lanes & d) == 0
        v, i = _stage(v, i, d, keep_high, lanes)
        d //= 2
    return v, i


def _score_topk_body(x_ref, w_ref, b_ref, probs_ref, idx_ref, scores_scr,
                     rv_scr, ri_scr):
    j = pl.program_id(0)
    xb = x_ref[...].astype(jnp.bfloat16).astype(jnp.float32)
    wv = w_ref[...].astype(jnp.bfloat16).astype(jnp.float32)
    s = jnp.sum(xb * wv, axis=-1) + b_ref[0, 0]
    scores_scr[:, pl.ds(j * BN, BN)] = s

    @pl.when(j == 0)
    def _init():
        rv_scr[...] = jnp.full((B, CAND), -jnp.inf, jnp.float32)
        ri_scr[...] = jnp.full((B, CAND), PADI, jnp.int32)

    bi = j * BN + lax.broadcasted_iota(jnp.int32, (B, BN), 1)
    bv, bi = _sort_rev(s, bi)
    fv = jnp.concatenate(
        [jnp.full((B, CAND - BN), -jnp.inf, jnp.float32), bv], axis=1)
    fi = jnp.concatenate(
        [jnp.full((B, CAND - BN), PADI, jnp.int32), bi], axis=1)
    rv, ri = rv_scr[...], ri_scr[...]
    g = _first(rv, ri, fv, fi)
    uv, ui = jnp.where(g, rv, fv), jnp.where(g, ri, fi)
    uv, ui = _halfclean_desc(uv, ui)
    rv_scr[...] = uv
    ri_scr[...] = ui

    @pl.when(j == NSTEPS - 1)
    def _finalize():
        scores = scores_scr[...]
        m = jnp.max(scores, axis=1, keepdims=True)
        u = jnp.exp(scores - m)
        ssum = jnp.sum(u, axis=1, keepdims=True)
        probs_ref[...] = u / ssum

        cv, ci = rv_scr[...], ri_scr[...]
        cp = jnp.exp(cv - m) / ssum
        lanes = lax.broadcasted_iota(jnp.int32, (B, CAND), 1)
        for q in range(REPAIR_PASSES):
            par = q % 2
            is_left = (lanes % 2) == par
            if par == 0:
                valid = jnp.full((B, CAND), True)
            else:
                valid = (lanes > 0) & (lanes < CAND - 1)
            pleft, pright = _rolls(cp, 1)
            pp = jnp.where(is_left, pleft, pright)
            ileft, iright = _rolls(ci, 1)
            pi = jnp.where(is_left, ileft, iright)
            eq = (cp == pp) & valid
            take = eq & ((is_left & (ci > pi)) | (~is_left & (ci < pi)))
            ci = jnp.where(take, pi, ci)
        idx_ref[...] = ci[:, :K]


_score_topk = pl.pallas_call(
    _score_topk_body,
    grid=(NSTEPS,),
    in_specs=[
        pl.BlockSpec((B, BN, D), lambda j: (0, j, 0)),
        pl.BlockSpec((D,), lambda j: (0,)),
        pl.BlockSpec(memory_space=pltpu.SMEM),
    ],
    out_specs=[
        pl.BlockSpec((B, N), lambda j: (0, 0)),
        pl.BlockSpec((B, K), lambda j: (0, 0)),
    ],
    out_shape=[
        jax.ShapeDtypeStruct((B, N), jnp.float32),
        jax.ShapeDtypeStruct((B, K), jnp.int32),
    ],
    scratch_shapes=[pltpu.VMEM((B, N), jnp.float32),
                    pltpu.VMEM((B, CAND), jnp.float32),
                    pltpu.VMEM((B, CAND), jnp.int32)],
)


_NC = 2
_NS = 16
_NW = _NC * _NS
_ROWS = B * K
_RPW = _ROWS // _NW


def _gather_body(x_hbm, idx_hbm, out_hbm, idx_v, rows_v, sem):
    wid = lax.axis_index("s") * _NC + lax.axis_index("c")
    base = wid * _RPW
    pltpu.sync_copy(idx_hbm.at[pl.ds(base, _RPW)], idx_v)
    pltpu.async_copy(x_hbm.at[idx_v], rows_v, sem).wait()
    pltpu.sync_copy(rows_v, out_hbm.at[pl.ds(base, _RPW)])


@functools.lru_cache(maxsize=None)
def _make_gather():
    return pl.kernel(
        _gather_body,
        out_type=jax.ShapeDtypeStruct((_ROWS, D), jnp.float32),
        mesh=plsc.VectorSubcoreMesh(core_axis_name="c", subcore_axis_name="s",
                                    num_cores=_NC, num_subcores=_NS),
        scratch_types=[
            pltpu.VMEM((_RPW,), jnp.int32),
            pltpu.VMEM((_RPW, D), jnp.float32),
            pltpu.SemaphoreType.DMA,
        ],
    )


def kernel(x, W, b):
    probs, idx = _score_topk(x, W, jnp.asarray(b).reshape(1, 1))
    flat = (idx + N * jnp.arange(B, dtype=jnp.int32)[:, None]).reshape(_ROWS)
    rows = _make_gather()(x.reshape(B * N, D), flat)
    selected = rows.reshape(B, K, D)
    return (selected, probs, idx)


# speedup vs baseline: 5.2316x; 5.2316x over previous
import functools

import jax
import jax.numpy as jnp
from jax import lax
from jax.experimental import pallas as pl
from jax.experimental.pallas import tpu as pltpu
from jax.experimental.pallas import tpu_sc as plsc

B, N, D, K = 4, 8192, 2048, 256
BN = 256
NSTEPS = N // BN
H = N // 2


def _swap4(x):
    return jnp.concatenate([x[4:8], x[0:4]], axis=0)


def _score_topk_body(x_ref, w_ref, b_ref, probs_ref, idx_ref, s8_scr):
    j = pl.program_id(0)
    xb = x_ref[...].astype(jnp.bfloat16).astype(jnp.float32)
    wv = w_ref[...].astype(jnp.bfloat16).astype(jnp.float32)
    s = jnp.sum(xb * wv, axis=-1) + b_ref[0, 0]

    @pl.when(j < NSTEPS // 2)
    def _store_lo():
        s8_scr[0:4, pl.ds(pl.multiple_of(j * BN, BN), BN)] = s

    @pl.when(j >= NSTEPS // 2)
    def _store_hi():
        s8_scr[4:8, pl.ds(pl.multiple_of((j - NSTEPS // 2) * BN, BN), BN)] = s

    @pl.when(j == NSTEPS - 1)
    def _finalize():
        w8 = s8_scr[...]
        m8 = jnp.max(w8, axis=1, keepdims=True)
        mb = jnp.maximum(m8, _swap4(m8))
        u8 = jnp.exp(w8 - mb)
        t8 = jnp.sum(u8, axis=1, keepdims=True)
        tb = t8 + _swap4(t8)
        p8 = u8 / tb
        probs_ref[:, 0:H] = p8[0:4]
        probs_ref[:, H:N] = p8[4:8]

        gl = lax.broadcasted_iota(jnp.int32, (8, H), 1)
        gh = jnp.where(lax.broadcasted_iota(jnp.int32, (8, H), 0) >= 4,
                       H, 0)
        g = gl + gh
        iota_k = lax.broadcasted_iota(jnp.int32, (8, K), 1)
        neg_inf = jnp.float32(-jnp.inf)

        def body(t, carry):
            v, acc = carry
            r8 = jnp.max(v, axis=1, keepdims=True)
            rb = jnp.maximum(r8, _swap4(r8))
            c = jnp.where(v == rb, g, N)
            i8 = jnp.min(c, axis=1, keepdims=True)
            ib = jnp.minimum(i8, _swap4(i8))
            acc = jnp.where(iota_k == t, ib, acc)
            v = jnp.where(g == ib, neg_inf, v)
            return v, acc

        acc0 = jnp.zeros((8, K), jnp.int32)
        _, acc = lax.fori_loop(0, K, body, (p8, acc0))
        idx_ref[...] = acc[0:4]


_score_topk = pl.pallas_call(
    _score_topk_body,
    grid=(NSTEPS,),
    in_specs=[
        pl.BlockSpec((B, BN, D), lambda j: (0, j, 0)),
        pl.BlockSpec((D,), lambda j: (0,)),
        pl.BlockSpec(memory_space=pltpu.SMEM),
    ],
    out_specs=[
        pl.BlockSpec((B, N), lambda j: (0, 0)),
        pl.BlockSpec((B, K), lambda j: (0, 0)),
    ],
    out_shape=[
        jax.ShapeDtypeStruct((B, N), jnp.float32),
        jax.ShapeDtypeStruct((B, K), jnp.int32),
    ],
    scratch_shapes=[pltpu.VMEM((8, H), jnp.float32)],
)


_NC = 2
_NS = 16
_NW = _NC * _NS
_ROWS = B * K
_RPW = _ROWS // _NW


def _gather_body(x_hbm, idx_hbm, out_hbm, idx_v, rows_v, sem):
    wid = lax.axis_index("s") * _NC + lax.axis_index("c")
    base = wid * _RPW
    pltpu.sync_copy(idx_hbm.at[pl.ds(base, _RPW)], idx_v)
    pltpu.async_copy(x_hbm.at[idx_v], rows_v, sem).wait()
    pltpu.sync_copy(rows_v, out_hbm.at[pl.ds(base, _RPW)])


@functools.lru_cache(maxsize=None)
def _make_gather():
    return pl.kernel(
        _gather_body,
        out_type=jax.ShapeDtypeStruct((_ROWS, D), jnp.float32),
        mesh=plsc.VectorSubcoreMesh(core_axis_name="c", subcore_axis_name="s",
                                    num_cores=_NC, num_subcores=_NS),
        scratch_types=[
            pltpu.VMEM((_RPW,), jnp.int32),
            pltpu.VMEM((_RPW, D), jnp.float32),
            pltpu.SemaphoreType.DMA,
        ],
    )


def kernel(x, W, b):
    probs, idx = _score_topk(x, W, jnp.asarray(b).reshape(1, 1))
    flat = (idx + N * jnp.arange(B, dtype=jnp.int32)[:, None]).reshape(_ROWS)
    rows = _make_gather()(x.reshape(B * N, D), flat)
    selected = rows.reshape(B, K, D)
    return (selected, probs, idx)


# speedup vs baseline: 5.7608x; 1.1012x over previous
import functools

import jax
import jax.numpy as jnp
from jax import lax
from jax.experimental import pallas as pl
from jax.experimental.pallas import tpu as pltpu
from jax.experimental.pallas import tpu_sc as plsc

B, N, D, K = 4, 8192, 2048, 256
BN = 256
NSTEPS = N // BN
def _score_topk_body(x_ref, w_ref, b_ref, probs_ref, idx_ref, s8_scr):
    j = pl.program_id(0)
    xb = x_ref[...].astype(jnp.bfloat16).astype(jnp.float32)
    wv = w_ref[...].astype(jnp.bfloat16).astype(jnp.float32)
    s = jnp.sum(xb * wv, axis=-1) + b_ref[0, 0]

    s8_scr[:, pl.ds(pl.multiple_of(j * BN, BN), BN)] = s

    @pl.when(j == NSTEPS - 1)
    def _finalize():
        scores = s8_scr[...]
        m = jnp.max(scores, axis=1, keepdims=True)
        u = jnp.exp(scores - m)
        ssum = jnp.sum(u, axis=1, keepdims=True)
        p = u / ssum
        probs_ref[...] = p

        iota = lax.broadcasted_iota(jnp.int32, (B, N), 1)
        iota_k = lax.broadcasted_iota(jnp.int32, (B, K), 1)
        neg_inf = jnp.float32(-jnp.inf)

        def body(t, carry):
            vals, acc = carry
            mx = jnp.max(vals, axis=1, keepdims=True)
            idx = jnp.min(jnp.where(vals == mx, iota, N), axis=1,
                          keepdims=True)
            acc = jnp.where(iota_k == t, idx, acc)
            vals = jnp.where(iota == idx, neg_inf, vals)
            return vals, acc

        acc0 = jnp.zeros((B, K), jnp.int32)
        _, acc = lax.fori_loop(0, K, body, (p, acc0))
        idx_ref[...] = acc


_score_topk = pl.pallas_call(
    _score_topk_body,
    grid=(NSTEPS,),
    in_specs=[
        pl.BlockSpec((B, BN, D), lambda j: (0, j, 0)),
        pl.BlockSpec((D,), lambda j: (0,)),
        pl.BlockSpec(memory_space=pltpu.SMEM),
    ],
    out_specs=[
        pl.BlockSpec((B, N), lambda j: (0, 0)),
        pl.BlockSpec((B, K), lambda j: (0, 0)),
    ],
    out_shape=[
        jax.ShapeDtypeStruct((B, N), jnp.float32),
        jax.ShapeDtypeStruct((B, K), jnp.int32),
    ],
    scratch_shapes=[pltpu.VMEM((B, N), jnp.float32)],
)


_NC = 2
_NS = 16
_NW = _NC * _NS
_ROWS = B * K
_RPW = _ROWS // _NW


def _gather_body(x_hbm, idx_hbm, out_hbm, idx_v, rows_v, sem):
    wid = lax.axis_index("s") * _NC + lax.axis_index("c")
    base = wid * _RPW
    pltpu.sync_copy(idx_hbm.at[pl.ds(base, _RPW)], idx_v)
    pltpu.async_copy(x_hbm.at[idx_v], rows_v, sem).wait()
    pltpu.sync_copy(rows_v, out_hbm.at[pl.ds(base, _RPW)])


@functools.lru_cache(maxsize=None)
def _make_gather():
    return pl.kernel(
        _gather_body,
        out_type=jax.ShapeDtypeStruct((_ROWS, D), jnp.float32),
        mesh=plsc.VectorSubcoreMesh(core_axis_name="c", subcore_axis_name="s",
                                    num_cores=_NC, num_subcores=_NS),
        scratch_types=[
            pltpu.VMEM((_RPW,), jnp.int32),
            pltpu.VMEM((_RPW, D), jnp.float32),
            pltpu.SemaphoreType.DMA,
        ],
    )


def kernel(x, W, b):
    probs, idx = _score_topk(x, W, jnp.asarray(b).reshape(1, 1))
    flat = (idx + N * jnp.arange(B, dtype=jnp.int32)[:, None]).reshape(_ROWS)
    rows = _make_gather()(x.reshape(B * N, D), flat)
    selected = rows.reshape(B, K, D)
    return (selected, probs, idx)
